# baseline (speedup 1.0000x reference)
import functools

import jax
import jax.numpy as jnp
from jax import lax
from jax.experimental import pallas as pl
from jax.experimental.pallas import tpu as pltpu
from jax.experimental.pallas import tpu_sc as plsc

LATENT = 64
BATCH = 16384
IDX_CHUNK = 128


def _mesh():
    return plsc.VectorSubcoreMesh(core_axis_name="c", subcore_axis_name="s")


def _gather_sc(nc, ns):
    nw = nc * ns
    bpw = BATCH // nw
    nchunk = bpw // IDX_CHUNK

    @functools.partial(
        pl.kernel,
        mesh=_mesh(),
        out_type=jax.ShapeDtypeStruct((BATCH * LATENT,), jnp.float32),
        compiler_params=pltpu.CompilerParams(use_tc_tiling_on_sc=True),
        scratch_types=[
            pltpu.VMEM((nchunk, IDX_CHUNK), jnp.int32),
            pltpu.VMEM((nchunk, IDX_CHUNK), jnp.int32),
            pltpu.VMEM((bpw, 2 * LATENT), jnp.float32),
            pltpu.VMEM((bpw * LATENT,), jnp.float32),
            pltpu.SemaphoreType.DMA,
        ],
    )
    def gather(idx_hbm, par_hbm, table_hbm, out_hbm,
               idx_v, par_v, rows_v, flat_v, sem):
        wid = lax.axis_index("s") * nc + lax.axis_index("c")
        base = wid * bpw
        pltpu.sync_copy(idx_hbm.at[wid], idx_v)
        pltpu.sync_copy(par_hbm.at[wid], par_v)
        copies = [
            pltpu.async_copy(table_hbm.at[idx_v.at[c]],
                             rows_v.at[pl.ds(c * IDX_CHUNK, IDX_CHUNK)], sem)
            for c in range(nchunk)
        ]
        for cp in copies:
            cp.wait()

        def group(g, carry):
            gbase = g * 16
            pv = par_v.at[g // (IDX_CHUNK // 16)][
                pl.ds((g % (IDX_CHUNK // 16)) * 16, 16)]
            for jj in range(16):
                j = gbase + jj
                off = pv[jj]
                for k in range(LATENT // 16):
                    flat_v[pl.ds(j * LATENT + k * 16, 16)] = (
                        rows_v[j, pl.ds(off + k * 16, 16)])
            return carry

        lax.fori_loop(0, bpw // 16, group, 0)
        pltpu.sync_copy(flat_v, out_hbm.at[pl.ds(base * LATENT, bpw * LATENT)])

    return gather


def _combine_sc(nc, ns):
    nw = nc * ns
    bpw = BATCH // nw
    ngroup = bpw // 16

    @functools.partial(
        pl.kernel,
        mesh=_mesh(),
        out_type=jax.ShapeDtypeStruct((BATCH,), jnp.float32),
        compiler_params=pltpu.CompilerParams(use_tc_tiling_on_sc=True),
        scratch_types=[
            pltpu.VMEM((bpw * LATENT,), jnp.float32),
            pltpu.VMEM((bpw * LATENT,), jnp.float32),
            pltpu.VMEM((LATENT,), jnp.float32),
            pltpu.VMEM((16,), jnp.float32),
            pltpu.VMEM((bpw,), jnp.float32),
            pltpu.SemaphoreType.DMA,
        ],
    )
    def combine(ru_hbm, ri_hbm, w_hbm, b_hbm, out_hbm,
                rows_u, rows_i, w_v, b_v, out_v, sem):
        wid = lax.axis_index("s") * nc + lax.axis_index("c")
        base = wid * bpw
        cu = pltpu.async_copy(ru_hbm.at[pl.ds(base * LATENT, bpw * LATENT)],
                              rows_u, sem)
        ci = pltpu.async_copy(ri_hbm.at[pl.ds(base * LATENT, bpw * LATENT)],
                              rows_i, sem)
        pltpu.sync_copy(w_hbm, w_v)
        pltpu.sync_copy(b_hbm, b_v)
        cu.wait()
        ci.wait()

        w0 = w_v[pl.ds(0, 16)]
        w1 = w_v[pl.ds(16, 16)]
        w2 = w_v[pl.ds(32, 16)]
        w3 = w_v[pl.ds(48, 16)]
        bias = b_v[...]
        lane = lax.iota(jnp.int32, 16)
        gd = lax.GatherDimensionNumbers(
            offset_dims=(), collapsed_slice_dims=(0,), start_index_map=(0,))

        def vperm(x, idx):
            return lax.gather(x, idx[:, None], gd, slice_sizes=(1,),
                              mode=lax.GatherScatterMode.PROMISE_IN_BOUNDS)

        def hsum_all(p):
            for bit in (8, 4, 2, 1):
                p = p + vperm(p, lane ^ bit)
            return p

        def group(g, carry):
            gbase = g * 16
            acc = jnp.zeros((16,), jnp.float32)
            for jj in range(16):
                j = (gbase + jj) * LATENT
                p = (rows_u[pl.ds(j, 16)] * rows_i[pl.ds(j, 16)] * w0
                     + rows_u[pl.ds(j + 16, 16)] * rows_i[pl.ds(j + 16, 16)] * w1
                     + rows_u[pl.ds(j + 32, 16)] * rows_i[pl.ds(j + 32, 16)] * w2
                     + rows_u[pl.ds(j + 48, 16)] * rows_i[pl.ds(j + 48, 16)] * w3)
                s = hsum_all(p)
                acc = jnp.where(lane == jj, s, acc)
            r = acc + bias
            r = 1.0 / (1.0 + jnp.exp(-r))
            out_v[pl.ds(gbase, 16)] = r
            return carry

        lax.fori_loop(0, ngroup, group, 0)
        pltpu.sync_copy(out_v, out_hbm.at[pl.ds(base, bpw)])

    return combine


def kernel(users, items, user_table, item_table, W, b):
    info = plsc.get_sparse_core_info()
    nc, ns = info.num_cores, info.num_subcores
    nw = nc * ns
    cshape = (nw, BATCH // nw // IDX_CHUNK, IDX_CHUNK)
    users = users.astype(jnp.int32)
    items = items.astype(jnp.int32)
    u_rows = (users >> 1).reshape(cshape)
    u_par = ((users & 1) * LATENT).reshape(cshape)
    i_rows = (items >> 1).reshape(cshape)
    i_par = ((items & 1) * LATENT).reshape(cshape)
    ut2 = user_table.reshape(NUM_ROWS2, 2 * LATENT)
    it2 = item_table.reshape(NUM_ROWS2, 2 * LATENT)
    gather = _gather_sc(nc, ns)
    rows_u = gather(u_rows, u_par, ut2)
    rows_i = gather(i_rows, i_par, it2)
    out = _combine_sc(nc, ns)(rows_u, rows_i, W.reshape(LATENT),
                              jnp.broadcast_to(b, (16,)))
    return out.reshape(BATCH, 1)


NUM_ROWS2 = 500000


# speedup vs baseline: 1.4990x; 1.4990x over previous
import functools

import jax
import jax.numpy as jnp
from jax import lax
from jax.experimental import pallas as pl
from jax.experimental.pallas import tpu as pltpu
from jax.experimental.pallas import tpu_sc as plsc

LATENT = 64
BATCH = 16384
NROWS = 1000000
CHUNK = 64


def _mesh():
    return plsc.VectorSubcoreMesh(core_axis_name="c", subcore_axis_name="s")


def _gather_sc(nc, ns):
    nw = nc * ns
    bpw = BATCH // nw

    @functools.partial(
        pl.kernel,
        mesh=_mesh(),
        out_type=jax.ShapeDtypeStruct((BATCH * LATENT,), jnp.float32),
        compiler_params=pltpu.CompilerParams(use_tc_tiling_on_sc=True),
        scratch_types=[
            pltpu.VMEM((bpw // 16, 16), jnp.int32),
            pltpu.VMEM((CHUNK, 8, LATENT), jnp.float32),
            pltpu.VMEM((bpw * LATENT,), jnp.float32),
            pltpu.SemaphoreType.DMA,
        ],
    )
    def gather(idx_hbm, table_hbm, out_hbm, idx_v, grp_v, flat_v, sem):
        wid = lax.axis_index("s") * nc + lax.axis_index("c")
        base = wid * bpw
        pltpu.sync_copy(idx_hbm.at[wid], idx_v)

        def row_src(r):
            rb = pl.multiple_of((r >> 3) << 3, 8)
            return table_hbm.at[pl.ds(rb, 8), :]

        def wave(wv, carry):
            wbase = wv * CHUNK

            def fire(g, c):
                vu = idx_v[(wbase + g * 16) // 16]
                for jj in range(16):
                    pltpu.async_copy(row_src(vu[jj]),
                                     grp_v.at[g * 16 + jj], sem)
                return c

            lax.fori_loop(0, CHUNK // 16, fire, 0)

            def drain(g, c):
                vu = idx_v[(wbase + g * 16) // 16]
                for jj in range(16):
                    pltpu.make_async_copy(row_src(vu[jj]),
                                          grp_v.at[g * 16 + jj], sem).wait()
                return c

            lax.fori_loop(0, CHUNK // 16, drain, 0)

            def extract(g, c):
                vu = idx_v[(wbase + g * 16) // 16]
                for jj in range(16):
                    rm = vu[jj] & 7
                    e = wbase + g * 16 + jj
                    for k in range(LATENT // 16):
                        flat_v[pl.ds(e * LATENT + k * 16, 16)] = (
                            grp_v[g * 16 + jj, rm, pl.ds(k * 16, 16)])
                return c

            lax.fori_loop(0, CHUNK // 16, extract, 0)
            return carry

        lax.fori_loop(0, bpw // CHUNK, wave, 0)
        pltpu.sync_copy(flat_v, out_hbm.at[pl.ds(base * LATENT, bpw * LATENT)])

    return gather


def _combine_sc(nc, ns):
    nw = nc * ns
    bpw = BATCH // nw
    ngroup = bpw // 16

    @functools.partial(
        pl.kernel,
        mesh=_mesh(),
        out_type=jax.ShapeDtypeStruct((BATCH,), jnp.float32),
        compiler_params=pltpu.CompilerParams(use_tc_tiling_on_sc=True),
        scratch_types=[
            pltpu.VMEM((bpw * LATENT,), jnp.float32),
            pltpu.VMEM((bpw * LATENT,), jnp.float32),
            pltpu.VMEM((LATENT,), jnp.float32),
            pltpu.VMEM((16,), jnp.float32),
            pltpu.VMEM((bpw,), jnp.float32),
            pltpu.SemaphoreType.DMA,
        ],
    )
    def combine(ru_hbm, ri_hbm, w_hbm, b_hbm, out_hbm,
                rows_u, rows_i, w_v, b_v, out_v, sem):
        wid = lax.axis_index("s") * nc + lax.axis_index("c")
        base = wid * bpw
        cu = pltpu.async_copy(ru_hbm.at[pl.ds(base * LATENT, bpw * LATENT)],
                              rows_u, sem)
        ci = pltpu.async_copy(ri_hbm.at[pl.ds(base * LATENT, bpw * LATENT)],
                              rows_i, sem)
        pltpu.sync_copy(w_hbm, w_v)
        pltpu.sync_copy(b_hbm, b_v)
        cu.wait()
        ci.wait()

        w0 = w_v[pl.ds(0, 16)]
        w1 = w_v[pl.ds(16, 16)]
        w2 = w_v[pl.ds(32, 16)]
        w3 = w_v[pl.ds(48, 16)]
        bias = b_v[...]
        lane = lax.iota(jnp.int32, 16)
        gd = lax.GatherDimensionNumbers(
            offset_dims=(), collapsed_slice_dims=(0,), start_index_map=(0,))

        def vperm(x, idx):
            return lax.gather(x, idx[:, None], gd, slice_sizes=(1,),
                              mode=lax.GatherScatterMode.PROMISE_IN_BOUNDS)

        def hsum_all(p):
            for bit in (8, 4, 2, 1):
                p = p + vperm(p, lane ^ bit)
            return p

        def group(g, carry):
            gbase = g * 16
            acc = jnp.zeros((16,), jnp.float32)
            for jj in range(16):
                j = (gbase + jj) * LATENT
                p = (rows_u[pl.ds(j, 16)] * rows_i[pl.ds(j, 16)] * w0
                     + rows_u[pl.ds(j + 16, 16)] * rows_i[pl.ds(j + 16, 16)] * w1
                     + rows_u[pl.ds(j + 32, 16)] * rows_i[pl.ds(j + 32, 16)] * w2
                     + rows_u[pl.ds(j + 48, 16)] * rows_i[pl.ds(j + 48, 16)] * w3)
                s = hsum_all(p)
                acc = jnp.where(lane == jj, s, acc)
            r = acc + bias
            r = 1.0 / (1.0 + jnp.exp(-r))
            out_v[pl.ds(gbase, 16)] = r
            return carry

        lax.fori_loop(0, ngroup, group, 0)
        pltpu.sync_copy(out_v, out_hbm.at[pl.ds(base, bpw)])

    return combine


def kernel(users, items, user_table, item_table, W, b):
    info = plsc.get_sparse_core_info()
    nc, ns = info.num_cores, info.num_subcores
    nw = nc * ns
    cshape = (nw, BATCH // nw // 16, 16)
    u3 = users.astype(jnp.int32).reshape(cshape)
    i3 = items.astype(jnp.int32).reshape(cshape)
    gather = _gather_sc(nc, ns)
    rows_u = gather(u3, user_table)
    rows_i = gather(i3, item_table)
    out = _combine_sc(nc, ns)(rows_u, rows_i, W.reshape(LATENT),
                              jnp.broadcast_to(b, (16,)))
    return out.reshape(BATCH, 1)


# speedup vs baseline: 1.5501x; 1.0341x over previous
import functools

import jax
import jax.numpy as jnp
from jax import lax
from jax.experimental import pallas as pl
from jax.experimental.pallas import tpu as pltpu
from jax.experimental.pallas import tpu_sc as plsc

LATENT = 64
BATCH = 16384
NROWS = 1000000
NB = 7813
RB = 245
TAIL_C = 7812
TAIL_W = NROWS - TAIL_C * 128
TRASH = BATCH


def _mesh():
    return plsc.VectorSubcoreMesh(core_axis_name="c", subcore_axis_name="s")


def _phase_a(nc, ns):
    @functools.partial(
        pl.kernel,
        mesh=_mesh(),
        out_type=(jax.ShapeDtypeStruct((BATCH + 1, 128), jnp.float32),
                  jax.ShapeDtypeStruct((BATCH + 1, 128), jnp.float32)),
        compiler_params=pltpu.CompilerParams(use_tc_tiling_on_sc=True, needs_layout_passes=False),
        scratch_types=[
            pltpu.VMEM((4096,), jnp.int32),
            pltpu.VMEM((BATCH + 16,), jnp.int32),
            pltpu.VMEM((BATCH + 16,), jnp.int32),
            pltpu.VMEM((2, LATENT, 128), jnp.float32),
            pltpu.VMEM((128, 128), jnp.float32),
            pltpu.VMEM((128,), jnp.int32),
            pltpu.SMEM((256,), jnp.int32),
            pltpu.SMEM((256,), jnp.int32),
            pltpu.SemaphoreType.DMA,
        ],
    )
    def phase_a(users_hbm, items_hbm, ut_hbm, it_hbm, tu_hbm, ti_hbm,
                scru_hbm, scri_hbm,
                idxbuf, hits, srt, blocks, staging, jlist,
                counts_sm, cum_sm, sem):
        wid = lax.axis_index("s") * nc + lax.axis_index("c")
        lo = wid * RB
        nb = jnp.minimum(RB, NB - lo)
        rlo = lo * 128
        rhi = (lo + nb) * 128
        lane = lax.iota(jnp.int32, 16)
        gd = lax.GatherDimensionNumbers(
            offset_dims=(), collapsed_slice_dims=(0,), start_index_map=(0,))

        def vperm(x, idx):
            return lax.gather(x, idx[:, None], gd, slice_sizes=(1,),
                              mode=lax.GatherScatterMode.PROMISE_IN_BOUNDS)

        def prefix16(x):
            for k in (1, 2, 4, 8):
                sh = vperm(x, jnp.maximum(lane - k, 0))
                x = x + jnp.where(lane >= k, sh, 0)
            return x

        DUMP = BATCH + 8

        def run(table_hbm, tail_hbm, idx_hbm, scr_hbm):
            def chunk_scan(ch, cursor):
                pltpu.sync_copy(idx_hbm.at[pl.ds(ch * 4096, 4096)], idxbuf)

                def step(i, cur):
                    r16 = idxbuf[pl.ds(i * 16, 16)]
                    j16 = lane + (ch * 4096 + i * 16)
                    m = (r16 >= rlo) & (r16 < rhi)
                    payload = ((r16 - rlo) << 14) | j16
                    mi = jnp.where(m, 1, 0)
                    incl = prefix16(mi)
                    pos = jnp.where(m, cur + incl - mi, DUMP)
                    plsc.store_scatter(hits, [pos], payload)
                    return cur + incl[15]

                return lax.fori_loop(0, 4096 // 16, step, cursor)

            cursor = 0
            for ch in range(4):
                cursor = chunk_scan(ch, cursor)

            def zero(c, carry):
                counts_sm[c] = 0
                return carry

            lax.fori_loop(0, 256, zero, 0)

            def hist(h, carry):
                p = hits[pl.ds(h, 16)][0]
                c = p >> 21
                counts_sm[c] = counts_sm[c] + 1
                return carry

            lax.fori_loop(0, cursor, hist, 0)

            def prefix(c, run_):
                cum_sm[c] = run_
                return run_ + counts_sm[c]

            lax.fori_loop(0, 256, prefix, 0)

            def place(h, carry):
                posreg, payreg = carry
                p = hits[pl.ds(h, 16)][0]
                c = p >> 21
                pos = cum_sm[c]
                cum_sm[c] = pos + 1
                sl = h & 15
                posreg = jnp.where(lane == sl, pos, posreg)
                payreg = jnp.where(lane == sl, p, payreg)

                @pl.when((sl == 15) | (h == cursor - 1))
                def _():
                    plsc.store_scatter(
                        srt, [jnp.where(lane <= sl, posreg, DUMP)], payreg)

                return posreg, payreg

            zero16 = jnp.zeros((16,), jnp.int32)
            lax.fori_loop(0, cursor, place, (zero16, zero16))

            def fire(c_loc, b):
                cg = lo + c_loc

                @pl.when(cg < TAIL_C)
                def _():
                    off = pl.multiple_of(cg * 128, 128)
                    pltpu.async_copy(table_hbm.at[:, pl.ds(off, 128)],
                                     blocks.at[b], sem)

                @pl.when(cg == TAIL_C)
                def _():
                    pltpu.async_copy(tail_hbm, blocks.at[b], sem)

            def wait(c_loc, b):
                cg = lo + c_loc

                @pl.when(cg < TAIL_C)
                def _():
                    off = pl.multiple_of(cg * 128, 128)
                    pltpu.make_async_copy(table_hbm.at[:, pl.ds(off, 128)],
                                          blocks.at[b], sem).wait()

                @pl.when(cg == TAIL_C)
                def _():
                    pltpu.make_async_copy(tail_hbm, blocks.at[b], sem).wait()

            fire(0, 0)

            def block_body(c_loc, cur2):
                b = c_loc & 1
                wait(c_loc, b)

                @pl.when(c_loc + 1 < nb)
                def _():
                    fire(c_loc + 1, 1 - b)

                prev = cum_sm[jnp.maximum(c_loc - 1, 0)]
                hstart = jnp.where(c_loc == 0, 0, prev)
                hend = cum_sm[c_loc]

                def hit(h, cur):
                    p = srt[pl.ds(h, 16)][0]
                    l = (p >> 14) & 127
                    j = p & 16383
                    for k in range(LATENT // 16):
                        g = plsc.load_gather(
                            blocks.at[b], [lane + k * 16,
                                           jnp.full((16,), l, jnp.int32)])
                        staging[cur, pl.ds(k * 16, 16)] = g
                    sl = cur & 15
                    jr = jlist[pl.ds((cur >> 4) * 16, 16)]
                    jr = jnp.where(lane == sl, j, jr)
                    jlist[pl.ds((cur >> 4) * 16, 16)] = jr

                    @pl.when(cur == 127)
                    def _():
                        pltpu.sync_copy(staging, scr_hbm.at[jlist])

                    return (cur + 1) & 127

                return lax.fori_loop(hstart, hend, hit, cur2)

            cur2 = lax.fori_loop(0, nb, block_body, 0)

            g0 = (cur2 + 15) >> 4

            @pl.when((cur2 & 15) != 0)
            def _():
                jr = jlist[pl.ds(((cur2 >> 4)) * 16, 16)]
                jr = jnp.where(lane < (cur2 & 15), jr, TRASH)
                jlist[pl.ds(((cur2 >> 4)) * 16, 16)] = jr

            def padg(g, carry):
                jlist[pl.ds(g * 16, 16)] = jnp.full((16,), TRASH, jnp.int32)
                return carry

            lax.fori_loop(g0, 8, padg, 0)

            @pl.when(cur2 > 0)
            def _():
                pltpu.sync_copy(staging, scr_hbm.at[jlist])

        run(ut_hbm, tu_hbm, users_hbm, scru_hbm)
        run(it_hbm, ti_hbm, items_hbm, scri_hbm)

    return phase_a


def _combine_sc(nc, ns):
    nw = nc * ns
    bpw = BATCH // nw
    WAVE = 256

    @functools.partial(
        pl.kernel,
        mesh=_mesh(),
        out_type=jax.ShapeDtypeStruct((BATCH,), jnp.float32),
        compiler_params=pltpu.CompilerParams(use_tc_tiling_on_sc=True, needs_layout_passes=False),
        scratch_types=[
            pltpu.VMEM((WAVE, 128), jnp.float32),
            pltpu.VMEM((WAVE, 128), jnp.float32),
            pltpu.VMEM((LATENT,), jnp.float32),
            pltpu.VMEM((16,), jnp.float32),
            pltpu.VMEM((bpw,), jnp.float32),
            pltpu.SemaphoreType.DMA,
        ],
    )
    def combine(ru_hbm, ri_hbm, w_hbm, b_hbm, out_hbm,
                rows_u, rows_i, w_v, b_v, out_v, sem):
        wid = lax.axis_index("s") * nc + lax.axis_index("c")
        base = wid * bpw
        pltpu.sync_copy(w_hbm, w_v)
        pltpu.sync_copy(b_hbm, b_v)

        w0 = w_v[pl.ds(0, 16)]
        w1 = w_v[pl.ds(16, 16)]
        w2 = w_v[pl.ds(32, 16)]
        w3 = w_v[pl.ds(48, 16)]
        bias = b_v[...]
        lane = lax.iota(jnp.int32, 16)
        gd = lax.GatherDimensionNumbers(
            offset_dims=(), collapsed_slice_dims=(0,), start_index_map=(0,))

        def vperm(x, idx):
            return lax.gather(x, idx[:, None], gd, slice_sizes=(1,),
                              mode=lax.GatherScatterMode.PROMISE_IN_BOUNDS)

        def hsum_all(p):
            for bit in (8, 4, 2, 1):
                p = p + vperm(p, lane ^ bit)
            return p

        for w in range(bpw // WAVE):
            cu = pltpu.async_copy(
                ru_hbm.at[pl.ds(base + w * WAVE, WAVE), :], rows_u, sem)
            ci = pltpu.async_copy(
                ri_hbm.at[pl.ds(base + w * WAVE, WAVE), :], rows_i, sem)
            cu.wait()
            ci.wait()

            def group(g, carry):
                gbase = g * 16
                acc = jnp.zeros((16,), jnp.float32)
                for jj in range(16):
                    j = gbase + jj
                    p = (rows_u[j, pl.ds(0, 16)] * rows_i[j, pl.ds(0, 16)] * w0
                         + rows_u[j, pl.ds(16, 16)]
                         * rows_i[j, pl.ds(16, 16)] * w1
                         + rows_u[j, pl.ds(32, 16)]
                         * rows_i[j, pl.ds(32, 16)] * w2
                         + rows_u[j, pl.ds(48, 16)]
                         * rows_i[j, pl.ds(48, 16)] * w3)
                    s = hsum_all(p)
                    acc = jnp.where(lane == jj, s, acc)
                r = acc + bias
                r = 1.0 / (1.0 + jnp.exp(-r))
                out_v[pl.ds(w * WAVE + gbase, 16)] = r
                return carry

            lax.fori_loop(0, WAVE // 16, group, 0)

        pltpu.sync_copy(out_v, out_hbm.at[pl.ds(base, bpw)])

    return combine


def kernel(users, items, user_table, item_table, W, b):
    info = plsc.get_sparse_core_info()
    nc, ns = info.num_cores, info.num_subcores
    users = users.astype(jnp.int32)
    items = items.astype(jnp.int32)
    tail_u = jnp.pad(user_table[TAIL_C * 128:, :].T, ((0, 0), (0, 128 - TAIL_W)))
    tail_i = jnp.pad(item_table[TAIL_C * 128:, :].T, ((0, 0), (0, 128 - TAIL_W)))
    scr_u, scr_i = _phase_a(nc, ns)(users, items, user_table.T, item_table.T,
                                    tail_u, tail_i)
    out = _combine_sc(nc, ns)(scr_u, scr_i, W.reshape(LATENT),
                              jnp.broadcast_to(b, (16,)))
    return out.reshape(BATCH, 1)


# speedup vs baseline: 2.2599x; 1.4579x over previous
import functools

import jax
import jax.numpy as jnp
from jax import lax
from jax.experimental import pallas as pl
from jax.experimental.pallas import tpu as pltpu
from jax.experimental.pallas import tpu_sc as plsc

LATENT = 64
BATCH = 16384
NROWS = 1000000
NB = 7813
RB = 245
TAIL_C = 7812
TAIL_W = NROWS - TAIL_C * 128
TRASH = BATCH


def _mesh():
    return plsc.VectorSubcoreMesh(core_axis_name="c", subcore_axis_name="s")


def _phase_a(nc, ns):
    @functools.partial(
        pl.kernel,
        mesh=_mesh(),
        out_type=(jax.ShapeDtypeStruct((BATCH + 1, 128), jnp.float32),
                  jax.ShapeDtypeStruct((BATCH + 1, 128), jnp.float32)),
        compiler_params=pltpu.CompilerParams(use_tc_tiling_on_sc=True, needs_layout_passes=False),
        scratch_types=[
            pltpu.VMEM((4096,), jnp.int32),
            pltpu.VMEM((BATCH + 16,), jnp.int32),
            pltpu.VMEM((BATCH + 16,), jnp.int32),
            pltpu.VMEM((4, LATENT, 128), jnp.float32),
            pltpu.VMEM((128, 128), jnp.float32),
            pltpu.VMEM((128,), jnp.int32),
            pltpu.SMEM((256,), jnp.int32),
            pltpu.SMEM((256,), jnp.int32),
            pltpu.SemaphoreType.DMA,
        ],
    )
    def phase_a(users_hbm, items_hbm, ut_hbm, it_hbm, tu_hbm, ti_hbm,
                scru_hbm, scri_hbm,
                idxbuf, hits, srt, blocks, staging, jlist,
                counts_sm, cum_sm, sem):
        wid = lax.axis_index("s") * nc + lax.axis_index("c")
        lo = wid * RB
        nb = jnp.minimum(RB, NB - lo)
        rlo = lo * 128
        rhi = (lo + nb) * 128
        lane = lax.iota(jnp.int32, 16)
        gd = lax.GatherDimensionNumbers(
            offset_dims=(), collapsed_slice_dims=(0,), start_index_map=(0,))

        def vperm(x, idx):
            return lax.gather(x, idx[:, None], gd, slice_sizes=(1,),
                              mode=lax.GatherScatterMode.PROMISE_IN_BOUNDS)

        def prefix16(x):
            for k in (1, 2, 4, 8):
                sh = vperm(x, jnp.maximum(lane - k, 0))
                x = x + jnp.where(lane >= k, sh, 0)
            return x

        DUMP = BATCH + 8

        def run(table_hbm, tail_hbm, idx_hbm, scr_hbm):
            def chunk_scan(ch, cursor):
                pltpu.sync_copy(idx_hbm.at[pl.ds(ch * 4096, 4096)], idxbuf)

                def step(i, cur):
                    r16 = idxbuf[pl.ds(i * 16, 16)]
                    j16 = lane + (ch * 4096 + i * 16)
                    m = (r16 >= rlo) & (r16 < rhi)
                    payload = ((r16 - rlo) << 14) | j16
                    mi = jnp.where(m, 1, 0)
                    incl = prefix16(mi)
                    pos = jnp.where(m, cur + incl - mi, DUMP)
                    plsc.store_scatter(hits, [pos], payload)
                    return cur + incl[15]

                return lax.fori_loop(0, 4096 // 16, step, cursor)

            cursor = 0
            for ch in range(4):
                cursor = chunk_scan(ch, cursor)

            def zero(c, carry):
                counts_sm[c] = 0
                return carry

            lax.fori_loop(0, 256, zero, 0)

            def hist(h, carry):
                p = hits[pl.ds(h, 16)][0]
                c = p >> 21
                counts_sm[c] = counts_sm[c] + 1
                return carry

            lax.fori_loop(0, cursor, hist, 0)

            def prefix(c, run_):
                cum_sm[c] = run_
                return run_ + counts_sm[c]

            lax.fori_loop(0, 256, prefix, 0)

            def place(h, carry):
                posreg, payreg = carry
                p = hits[pl.ds(h, 16)][0]
                c = p >> 21
                pos = cum_sm[c]
                cum_sm[c] = pos + 1
                sl = h & 15
                posreg = jnp.where(lane == sl, pos, posreg)
                payreg = jnp.where(lane == sl, p, payreg)

                @pl.when((sl == 15) | (h == cursor - 1))
                def _():
                    plsc.store_scatter(
                        srt, [jnp.where(lane <= sl, posreg, DUMP)], payreg)

                return posreg, payreg

            zero16 = jnp.zeros((16,), jnp.int32)
            lax.fori_loop(0, cursor, place, (zero16, zero16))

            def bounds(c_loc):
                prev = cum_sm[jnp.maximum(c_loc - 1, 0)]
                hstart = jnp.where(c_loc == 0, 0, prev)
                return hstart, cum_sm[c_loc]

            def fire(c_loc):
                cg = lo + c_loc
                b = c_loc & 3
                hs, he = bounds(c_loc)

                @pl.when((he > hs) & (cg < TAIL_C))
                def _():
                    off = pl.multiple_of(cg * 128, 128)
                    pltpu.async_copy(table_hbm.at[:, pl.ds(off, 128)],
                                     blocks.at[b], sem)

                @pl.when((he > hs) & (cg == TAIL_C))
                def _():
                    pltpu.async_copy(tail_hbm, blocks.at[b], sem)

            def wait(c_loc):
                cg = lo + c_loc
                b = c_loc & 3
                hs, he = bounds(c_loc)

                @pl.when((he > hs) & (cg < TAIL_C))
                def _():
                    off = pl.multiple_of(cg * 128, 128)
                    pltpu.make_async_copy(table_hbm.at[:, pl.ds(off, 128)],
                                          blocks.at[b], sem).wait()

                @pl.when((he > hs) & (cg == TAIL_C))
                def _():
                    pltpu.make_async_copy(tail_hbm, blocks.at[b], sem).wait()

            def prime(c_loc, carry):
                fire(c_loc)
                return carry

            lax.fori_loop(0, jnp.minimum(3, nb), prime, 0)

            def block_body(c_loc, cur2):
                b = c_loc & 3
                wait(c_loc)

                @pl.when(c_loc + 3 < nb)
                def _():
                    fire(c_loc + 3)

                hstart, hend = bounds(c_loc)

                def hit(h, cur):
                    p = srt[pl.ds(h, 16)][0]
                    l = (p >> 14) & 127
                    j = p & 16383
                    for k in range(LATENT // 16):
                        g = plsc.load_gather(
                            blocks.at[b], [lane + k * 16,
                                           jnp.full((16,), l, jnp.int32)])
                        staging[cur, pl.ds(k * 16, 16)] = g
                    sl = cur & 15
                    jr = jlist[pl.ds((cur >> 4) * 16, 16)]
                    jr = jnp.where(lane == sl, j, jr)
                    jlist[pl.ds((cur >> 4) * 16, 16)] = jr

                    @pl.when(cur == 127)
                    def _():
                        pltpu.sync_copy(staging, scr_hbm.at[jlist])

                    return (cur + 1) & 127

                return lax.fori_loop(hstart, hend, hit, cur2)

            cur2 = lax.fori_loop(0, nb, block_body, 0)

            g0 = (cur2 + 15) >> 4

            @pl.when((cur2 & 15) != 0)
            def _():
                jr = jlist[pl.ds(((cur2 >> 4)) * 16, 16)]
                jr = jnp.where(lane < (cur2 & 15), jr, TRASH)
                jlist[pl.ds(((cur2 >> 4)) * 16, 16)] = jr

            def padg(g, carry):
                jlist[pl.ds(g * 16, 16)] = jnp.full((16,), TRASH, jnp.int32)
                return carry

            lax.fori_loop(g0, 8, padg, 0)

            @pl.when(cur2 > 0)
            def _():
                pltpu.sync_copy(staging, scr_hbm.at[jlist])

        run(ut_hbm, tu_hbm, users_hbm, scru_hbm)
        run(it_hbm, ti_hbm, items_hbm, scri_hbm)

    return phase_a


def _combine_sc(nc, ns):
    nw = nc * ns
    bpw = BATCH // nw
    WAVE = 256

    @functools.partial(
        pl.kernel,
        mesh=_mesh(),
        out_type=jax.ShapeDtypeStruct((BATCH,), jnp.float32),
        compiler_params=pltpu.CompilerParams(use_tc_tiling_on_sc=True, needs_layout_passes=False),
        scratch_types=[
            pltpu.VMEM((WAVE, 128), jnp.float32),
            pltpu.VMEM((WAVE, 128), jnp.float32),
            pltpu.VMEM((LATENT,), jnp.float32),
            pltpu.VMEM((16,), jnp.float32),
            pltpu.VMEM((bpw,), jnp.float32),
            pltpu.SemaphoreType.DMA,
        ],
    )
    def combine(ru_hbm, ri_hbm, w_hbm, b_hbm, out_hbm,
                rows_u, rows_i, w_v, b_v, out_v, sem):
        wid = lax.axis_index("s") * nc + lax.axis_index("c")
        base = wid * bpw
        pltpu.sync_copy(w_hbm, w_v)
        pltpu.sync_copy(b_hbm, b_v)

        w0 = w_v[pl.ds(0, 16)]
        w1 = w_v[pl.ds(16, 16)]
        w2 = w_v[pl.ds(32, 16)]
        w3 = w_v[pl.ds(48, 16)]
        bias = b_v[...]
        lane = lax.iota(jnp.int32, 16)
        gd = lax.GatherDimensionNumbers(
            offset_dims=(), collapsed_slice_dims=(0,), start_index_map=(0,))

        def vperm(x, idx):
            return lax.gather(x, idx[:, None], gd, slice_sizes=(1,),
                              mode=lax.GatherScatterMode.PROMISE_IN_BOUNDS)

        def hsum_all(p):
            for bit in (8, 4, 2, 1):
                p = p + vperm(p, lane ^ bit)
            return p

        for w in range(bpw // WAVE):
            cu = pltpu.async_copy(
                ru_hbm.at[pl.ds(base + w * WAVE, WAVE), :], rows_u, sem)
            ci = pltpu.async_copy(
                ri_hbm.at[pl.ds(base + w * WAVE, WAVE), :], rows_i, sem)
            cu.wait()
            ci.wait()

            def group(g, carry):
                gbase = g * 16
                acc = jnp.zeros((16,), jnp.float32)
                for jj in range(16):
                    j = gbase + jj
                    p = (rows_u[j, pl.ds(0, 16)] * rows_i[j, pl.ds(0, 16)] * w0
                         + rows_u[j, pl.ds(16, 16)]
                         * rows_i[j, pl.ds(16, 16)] * w1
                         + rows_u[j, pl.ds(32, 16)]
                         * rows_i[j, pl.ds(32, 16)] * w2
                         + rows_u[j, pl.ds(48, 16)]
                         * rows_i[j, pl.ds(48, 16)] * w3)
                    s = hsum_all(p)
                    acc = jnp.where(lane == jj, s, acc)
                r = acc + bias
                r = 1.0 / (1.0 + jnp.exp(-r))
                out_v[pl.ds(w * WAVE + gbase, 16)] = r
                return carry

            lax.fori_loop(0, WAVE // 16, group, 0)

        pltpu.sync_copy(out_v, out_hbm.at[pl.ds(base, bpw)])

    return combine


def kernel(users, items, user_table, item_table, W, b):
    info = plsc.get_sparse_core_info()
    nc, ns = info.num_cores, info.num_subcores
    users = users.astype(jnp.int32)
    items = items.astype(jnp.int32)
    tail_u = jnp.pad(user_table[TAIL_C * 128:, :].T, ((0, 0), (0, 128 - TAIL_W)))
    tail_i = jnp.pad(item_table[TAIL_C * 128:, :].T, ((0, 0), (0, 128 - TAIL_W)))
    scr_u, scr_i = _phase_a(nc, ns)(users, items, user_table.T, item_table.T,
                                    tail_u, tail_i)
    out = _combine_sc(nc, ns)(scr_u, scr_i, W.reshape(LATENT),
                              jnp.broadcast_to(b, (16,)))
    return out.reshape(BATCH, 1)


# speedup vs baseline: 2.4260x; 1.0735x over previous
import functools

import jax
import jax.numpy as jnp
from jax import lax
from jax.experimental import pallas as pl
from jax.experimental.pallas import tpu as pltpu
from jax.experimental.pallas import tpu_sc as plsc

LATENT = 64
BATCH = 16384
NROWS = 1000000
NB = 7813
RB = 245
TAIL_C = 7812
TAIL_W = NROWS - TAIL_C * 128
TRASH = BATCH


def _mesh():
    return plsc.VectorSubcoreMesh(core_axis_name="c", subcore_axis_name="s")


def _phase_a(nc, ns):
    @functools.partial(
        pl.kernel,
        mesh=_mesh(),
        out_type=(jax.ShapeDtypeStruct((BATCH + 1, 128), jnp.float32),
                  jax.ShapeDtypeStruct((BATCH + 1, 128), jnp.float32)),
        compiler_params=pltpu.CompilerParams(use_tc_tiling_on_sc=True, needs_layout_passes=False),
        scratch_types=[
            pltpu.VMEM((4096,), jnp.int32),
            pltpu.VMEM((BATCH + 16,), jnp.int32),
            pltpu.VMEM((BATCH + 16,), jnp.int32),
            pltpu.VMEM((8, LATENT, 128), jnp.float32),
            pltpu.VMEM((128, 128), jnp.float32),
            pltpu.VMEM((128,), jnp.int32),
            pltpu.SMEM((256,), jnp.int32),
            pltpu.SMEM((256,), jnp.int32),
            pltpu.SemaphoreType.DMA,
        ],
    )
    def phase_a(users_hbm, items_hbm, ut_hbm, it_hbm, tu_hbm, ti_hbm,
                scru_hbm, scri_hbm,
                idxbuf, hits, srt, blocks, staging, jlist,
                counts_sm, cum_sm, sem):
        wid = lax.axis_index("s") * nc + lax.axis_index("c")
        lo = wid * RB
        nb = jnp.minimum(RB, NB - lo)
        rlo = lo * 128
        rhi = (lo + nb) * 128
        lane = lax.iota(jnp.int32, 16)
        gd = lax.GatherDimensionNumbers(
            offset_dims=(), collapsed_slice_dims=(0,), start_index_map=(0,))

        def vperm(x, idx):
            return lax.gather(x, idx[:, None], gd, slice_sizes=(1,),
                              mode=lax.GatherScatterMode.PROMISE_IN_BOUNDS)

        def prefix16(x):
            for k in (1, 2, 4, 8):
                sh = vperm(x, jnp.maximum(lane - k, 0))
                x = x + jnp.where(lane >= k, sh, 0)
            return x

        DUMP = BATCH + 8

        def run(table_hbm, tail_hbm, idx_hbm, scr_hbm):
            def chunk_scan(ch, cursor):
                pltpu.sync_copy(idx_hbm.at[pl.ds(ch * 4096, 4096)], idxbuf)

                def step(i, cur):
                    r16 = idxbuf[pl.ds(i * 16, 16)]
                    j16 = lane + (ch * 4096 + i * 16)
                    m = (r16 >= rlo) & (r16 < rhi)
                    payload = ((r16 - rlo) << 14) | j16
                    mi = jnp.where(m, 1, 0)
                    incl = prefix16(mi)
                    pos = jnp.where(m, cur + incl - mi, DUMP)
                    plsc.store_scatter(hits, [pos], payload)
                    return cur + incl[15]

                return lax.fori_loop(0, 4096 // 16, step, cursor)

            cursor = 0
            for ch in range(4):
                cursor = chunk_scan(ch, cursor)

            def zero(c, carry):
                counts_sm[c] = 0
                return carry

            lax.fori_loop(0, 256, zero, 0)

            def hist(h, carry):
                p = hits[pl.ds(h, 16)][0]
                c = p >> 21
                counts_sm[c] = counts_sm[c] + 1
                return carry

            lax.fori_loop(0, cursor, hist, 0)

            def prefix(c, run_):
                cum_sm[c] = run_
                return run_ + counts_sm[c]

            lax.fori_loop(0, 256, prefix, 0)

            def place(h, carry):
                posreg, payreg = carry
                p = hits[pl.ds(h, 16)][0]
                c = p >> 21
                pos = cum_sm[c]
                cum_sm[c] = pos + 1
                sl = h & 15
                posreg = jnp.where(lane == sl, pos, posreg)
                payreg = jnp.where(lane == sl, p, payreg)

                @pl.when((sl == 15) | (h == cursor - 1))
                def _():
                    plsc.store_scatter(
                        srt, [jnp.where(lane <= sl, posreg, DUMP)], payreg)

                return posreg, payreg

            zero16 = jnp.zeros((16,), jnp.int32)
            lax.fori_loop(0, cursor, place, (zero16, zero16))

            def bounds(c_loc):
                prev = cum_sm[jnp.maximum(c_loc - 1, 0)]
                hstart = jnp.where(c_loc == 0, 0, prev)
                return hstart, cum_sm[c_loc]

            def fire(c_loc):
                cg = lo + c_loc
                b = c_loc & 7
                hs, he = bounds(c_loc)

                @pl.when((he > hs) & (cg < TAIL_C))
                def _():
                    off = pl.multiple_of(cg * 128, 128)
                    pltpu.async_copy(table_hbm.at[:, pl.ds(off, 128)],
                                     blocks.at[b], sem)

                @pl.when((he > hs) & (cg == TAIL_C))
                def _():
                    pltpu.async_copy(tail_hbm, blocks.at[b], sem)

            def wait(c_loc):
                cg = lo + c_loc
                b = c_loc & 7
                hs, he = bounds(c_loc)

                @pl.when((he > hs) & (cg < TAIL_C))
                def _():
                    off = pl.multiple_of(cg * 128, 128)
                    pltpu.make_async_copy(table_hbm.at[:, pl.ds(off, 128)],
                                          blocks.at[b], sem).wait()

                @pl.when((he > hs) & (cg == TAIL_C))
                def _():
                    pltpu.make_async_copy(tail_hbm, blocks.at[b], sem).wait()

            def prime(c_loc, carry):
                fire(c_loc)
                return carry

            lax.fori_loop(0, jnp.minimum(7, nb), prime, 0)

            def block_body(c_loc, cur2):
                b = c_loc & 7
                wait(c_loc)

                @pl.when(c_loc + 7 < nb)
                def _():
                    fire(c_loc + 7)

                hstart, hend = bounds(c_loc)

                def hit(h, cur):
                    p = srt[pl.ds(h, 16)][0]
                    l = (p >> 14) & 127
                    j = p & 16383
                    for k in range(LATENT // 16):
                        g = plsc.load_gather(
                            blocks.at[b], [lane + k * 16,
                                           jnp.full((16,), l, jnp.int32)])
                        staging[cur, pl.ds(k * 16, 16)] = g
                    sl = cur & 15
                    jr = jlist[pl.ds((cur >> 4) * 16, 16)]
                    jr = jnp.where(lane == sl, j, jr)
                    jlist[pl.ds((cur >> 4) * 16, 16)] = jr

                    @pl.when(cur == 127)
                    def _():
                        pltpu.sync_copy(staging, scr_hbm.at[jlist])

                    return (cur + 1) & 127

                return lax.fori_loop(hstart, hend, hit, cur2)

            cur2 = lax.fori_loop(0, nb, block_body, 0)

            g0 = (cur2 + 15) >> 4

            @pl.when((cur2 & 15) != 0)
            def _():
                jr = jlist[pl.ds(((cur2 >> 4)) * 16, 16)]
                jr = jnp.where(lane < (cur2 & 15), jr, TRASH)
                jlist[pl.ds(((cur2 >> 4)) * 16, 16)] = jr

            def padg(g, carry):
                jlist[pl.ds(g * 16, 16)] = jnp.full((16,), TRASH, jnp.int32)
                return carry

            lax.fori_loop(g0, 8, padg, 0)

            @pl.when(cur2 > 0)
            def _():
                pltpu.sync_copy(staging, scr_hbm.at[jlist])

        run(ut_hbm, tu_hbm, users_hbm, scru_hbm)
        run(it_hbm, ti_hbm, items_hbm, scri_hbm)

    return phase_a


def _combine_sc(nc, ns):
    nw = nc * ns
    bpw = BATCH // nw
    WAVE = 256

    @functools.partial(
        pl.kernel,
        mesh=_mesh(),
        out_type=jax.ShapeDtypeStruct((BATCH,), jnp.float32),
        compiler_params=pltpu.CompilerParams(use_tc_tiling_on_sc=True, needs_layout_passes=False),
        scratch_types=[
            pltpu.VMEM((WAVE, 128), jnp.float32),
            pltpu.VMEM((WAVE, 128), jnp.float32),
            pltpu.VMEM((LATENT,), jnp.float32),
            pltpu.VMEM((16,), jnp.float32),
            pltpu.VMEM((bpw,), jnp.float32),
            pltpu.SemaphoreType.DMA,
        ],
    )
    def combine(ru_hbm, ri_hbm, w_hbm, b_hbm, out_hbm,
                rows_u, rows_i, w_v, b_v, out_v, sem):
        wid = lax.axis_index("s") * nc + lax.axis_index("c")
        base = wid * bpw
        pltpu.sync_copy(w_hbm, w_v)
        pltpu.sync_copy(b_hbm, b_v)

        w0 = w_v[pl.ds(0, 16)]
        w1 = w_v[pl.ds(16, 16)]
        w2 = w_v[pl.ds(32, 16)]
        w3 = w_v[pl.ds(48, 16)]
        bias = b_v[...]
        lane = lax.iota(jnp.int32, 16)
        gd = lax.GatherDimensionNumbers(
            offset_dims=(), collapsed_slice_dims=(0,), start_index_map=(0,))

        def vperm(x, idx):
            return lax.gather(x, idx[:, None], gd, slice_sizes=(1,),
                              mode=lax.GatherScatterMode.PROMISE_IN_BOUNDS)

        def hsum_all(p):
            for bit in (8, 4, 2, 1):
                p = p + vperm(p, lane ^ bit)
            return p

        for w in range(bpw // WAVE):
            cu = pltpu.async_copy(
                ru_hbm.at[pl.ds(base + w * WAVE, WAVE), :], rows_u, sem)
            ci = pltpu.async_copy(
                ri_hbm.at[pl.ds(base + w * WAVE, WAVE), :], rows_i, sem)
            cu.wait()
            ci.wait()

            def group(g, carry):
                gbase = g * 16
                acc = jnp.zeros((16,), jnp.float32)
                for jj in range(16):
                    j = gbase + jj
                    p = (rows_u[j, pl.ds(0, 16)] * rows_i[j, pl.ds(0, 16)] * w0
                         + rows_u[j, pl.ds(16, 16)]
                         * rows_i[j, pl.ds(16, 16)] * w1
                         + rows_u[j, pl.ds(32, 16)]
                         * rows_i[j, pl.ds(32, 16)] * w2
                         + rows_u[j, pl.ds(48, 16)]
                         * rows_i[j, pl.ds(48, 16)] * w3)
                    s = hsum_all(p)
                    acc = jnp.where(lane == jj, s, acc)
                r = acc + bias
                r = 1.0 / (1.0 + jnp.exp(-r))
                out_v[pl.ds(w * WAVE + gbase, 16)] = r
                return carry

            lax.fori_loop(0, WAVE // 16, group, 0)

        pltpu.sync_copy(out_v, out_hbm.at[pl.ds(base, bpw)])

    return combine


def kernel(users, items, user_table, item_table, W, b):
    info = plsc.get_sparse_core_info()
    nc, ns = info.num_cores, info.num_subcores
    users = users.astype(jnp.int32)
    items = items.astype(jnp.int32)
    tail_u = jnp.pad(user_table[TAIL_C * 128:, :].T, ((0, 0), (0, 128 - TAIL_W)))
    tail_i = jnp.pad(item_table[TAIL_C * 128:, :].T, ((0, 0), (0, 128 - TAIL_W)))
    scr_u, scr_i = _phase_a(nc, ns)(users, items, user_table.T, item_table.T,
                                    tail_u, tail_i)
    out = _combine_sc(nc, ns)(scr_u, scr_i, W.reshape(LATENT),
                              jnp.broadcast_to(b, (16,)))
    return out.reshape(BATCH, 1)
